# Initial kernel scaffold; baseline (speedup 1.0000x reference)
import jax
import jax.numpy as jnp
from jax.experimental import pallas as pl


def kernel(x, table, W, b):
    raise NotImplementedError("write your pallas kernel here")


# speedup vs baseline: 3.0886x; 3.0886x over previous
import functools

import jax
import jax.numpy as jnp
from jax import lax
from jax.experimental import pallas as pl
from jax.experimental.pallas import tpu as pltpu
from jax.experimental.pallas import tpu_sc as plsc

B_ROWS = 4096
SEQ = 20
D = 768
N_LOOKUPS = B_ROWS * SEQ

NC = 2
NS = 16
NW = NC * NS
ROWS_PER_W = N_LOOKUPS // NW
CHUNK = 128
NCHUNK = ROWS_PER_W // CHUNK


def _gather_body(x_hbm, table_hbm, out_hbm, idx_v, rows_v, sem):
    wid = lax.axis_index("s") * NC + lax.axis_index("c")
    base = wid * ROWS_PER_W

    def chunk(k, carry):
        off = base + k * CHUNK
        pltpu.sync_copy(x_hbm.at[pl.ds(off, CHUNK)], idx_v)
        pltpu.async_copy(table_hbm.at[idx_v], rows_v, sem).wait()
        pltpu.sync_copy(rows_v, out_hbm.at[pl.ds(off, CHUNK)])
        return carry

    lax.fori_loop(0, NCHUNK, chunk, 0)


def _sc_gather(x_flat, table):
    mesh = plsc.VectorSubcoreMesh(core_axis_name="c", subcore_axis_name="s")
    kern = functools.partial(
        pl.kernel,
        mesh=mesh,
        out_type=jax.ShapeDtypeStruct((N_LOOKUPS, D), jnp.float32),
        scratch_types=[
            pltpu.VMEM((CHUNK,), jnp.int32),
            pltpu.VMEM((CHUNK, D), jnp.float32),
            pltpu.SemaphoreType.DMA,
        ],
    )(_gather_body)
    return kern(x_flat, table)


_SQRT_HALF = 0.7071067811865476


def _mm_body(emb_ref, wt_ref, b_ref, out_ref):
    h = jnp.dot(emb_ref[...], wt_ref[...], preferred_element_type=jnp.float32)
    h = h + b_ref[...]
    out_ref[...] = 0.5 * h * (1.0 + lax.erf(h * _SQRT_HALF))


def _tc_linear_gelu(emb, wt, bias):
    BM = 1024
    grid = (N_LOOKUPS // BM,)
    return pl.pallas_call(
        _mm_body,
        grid=grid,
        in_specs=[
            pl.BlockSpec((BM, D), lambda i: (i, 0)),
            pl.BlockSpec((D, D), lambda i: (0, 0)),
            pl.BlockSpec((1, D), lambda i: (0, 0)),
        ],
        out_specs=pl.BlockSpec((BM, D), lambda i: (i, 0)),
        out_shape=jax.ShapeDtypeStruct((N_LOOKUPS, D), jnp.float32),
    )(emb, wt, bias)


def kernel(x, table, W, b):
    x_flat = x.reshape(-1).astype(jnp.int32)
    emb = _sc_gather(x_flat, table)
    out = _tc_linear_gelu(emb, W.T, b.reshape(1, D))
    return out.reshape(B_ROWS, SEQ, D)


# speedup vs baseline: 6.4708x; 2.0951x over previous
import functools

import jax
import jax.numpy as jnp
from jax import lax
from jax.experimental import pallas as pl
from jax.experimental.pallas import tpu as pltpu
from jax.experimental.pallas import tpu_sc as plsc

B_ROWS = 4096
SEQ = 20
D = 768
N_LOOKUPS = B_ROWS * SEQ

NC = 2
NS = 16
NW = NC * NS
ROWS_PER_W = N_LOOKUPS // NW
CHUNK = 128
NCHUNK = ROWS_PER_W // CHUNK


def _gather_body(x_hbm, table_hbm, out_hbm, idx_v, rows_v, sem):
    wid = lax.axis_index("s") * NC + lax.axis_index("c")
    base = wid * ROWS_PER_W

    def chunk(k, carry):
        off = base + k * CHUNK
        pltpu.sync_copy(x_hbm.at[pl.ds(off, CHUNK)], idx_v)
        pltpu.async_copy(table_hbm.at[idx_v], rows_v, sem).wait()
        pltpu.sync_copy(rows_v, out_hbm.at[pl.ds(off, CHUNK)])
        return carry

    lax.fori_loop(0, NCHUNK, chunk, 0)


def _sc_gather(x_flat, table):
    mesh = plsc.VectorSubcoreMesh(core_axis_name="c", subcore_axis_name="s")
    kern = functools.partial(
        pl.kernel,
        mesh=mesh,
        out_type=jax.ShapeDtypeStruct((N_LOOKUPS, D), jnp.float32),
        scratch_types=[
            pltpu.VMEM((CHUNK,), jnp.int32),
            pltpu.VMEM((CHUNK, D), jnp.float32),
            pltpu.SemaphoreType.DMA,
        ],
    )(_gather_body)
    return kern(x_flat, table)


_SQRT_HALF = 0.7071067811865476


def _mm_body(emb_ref, wt_ref, b_ref, out_ref):
    h = jnp.dot(emb_ref[...], wt_ref[...], preferred_element_type=jnp.float32)
    h = h + b_ref[...]
    out_ref[...] = 0.5 * h * (1.0 + lax.erf(h * _SQRT_HALF))


def _tc_linear_gelu(emb, wt, bias):
    BM = 1024
    grid = (N_LOOKUPS // BM,)
    return pl.pallas_call(
        _mm_body,
        grid=grid,
        in_specs=[
            pl.BlockSpec((BM, D), lambda i: (i, 0)),
            pl.BlockSpec((D, D), lambda i: (0, 0)),
            pl.BlockSpec((1, D), lambda i: (0, 0)),
        ],
        out_specs=pl.BlockSpec((BM, D), lambda i: (i, 0)),
        out_shape=jax.ShapeDtypeStruct((N_LOOKUPS, D), jnp.float32),
    )(emb, wt, bias)


def kernel(x, table, W, b):
    x_t = x.astype(jnp.int32).T.reshape(-1)
    emb = _sc_gather(x_t, table)
    out = _tc_linear_gelu(emb, W.T, b.reshape(1, D))
    return out.reshape(SEQ, B_ROWS, D).transpose(1, 0, 2)


# speedup vs baseline: 6.5284x; 1.0089x over previous
import functools

import jax
import jax.numpy as jnp
from jax import lax
from jax.experimental import pallas as pl
from jax.experimental.pallas import tpu as pltpu
from jax.experimental.pallas import tpu_sc as plsc

B_ROWS = 4096
SEQ = 20
D = 768
N_LOOKUPS = B_ROWS * SEQ

NC = 2
NS = 16
NW = NC * NS
ROWS_PER_W = N_LOOKUPS // NW
CHUNK = 128
NCHUNK = ROWS_PER_W // CHUNK


def _gather_body(x_hbm, table_hbm, out_hbm, idx_v, rows_v, sem):
    wid = lax.axis_index("s") * NC + lax.axis_index("c")
    base = wid * ROWS_PER_W

    def chunk(k, carry):
        off = base + k * CHUNK
        pltpu.sync_copy(x_hbm.at[pl.ds(off, CHUNK)], idx_v)
        pltpu.async_copy(table_hbm.at[idx_v], rows_v, sem).wait()
        pltpu.sync_copy(rows_v, out_hbm.at[pl.ds(off, CHUNK)])
        return carry

    lax.fori_loop(0, NCHUNK, chunk, 0)


def _sc_gather(x_flat, table):
    mesh = plsc.VectorSubcoreMesh(core_axis_name="c", subcore_axis_name="s")
    kern = functools.partial(
        pl.kernel,
        mesh=mesh,
        out_type=jax.ShapeDtypeStruct((N_LOOKUPS, D), jnp.float32),
        scratch_types=[
            pltpu.VMEM((CHUNK,), jnp.int32),
            pltpu.VMEM((CHUNK, D), jnp.float32),
            pltpu.SemaphoreType.DMA,
        ],
    )(_gather_body)
    return kern(x_flat, table)


_SQRT_HALF = 0.7071067811865476


def _mm_body(emb_ref, wt_ref, b_ref, out_ref):
    h = jnp.dot(emb_ref[...].astype(jnp.bfloat16), wt_ref[...],
                preferred_element_type=jnp.float32)
    h = h + b_ref[...]
    out_ref[...] = 0.5 * h * (1.0 + lax.erf(h * _SQRT_HALF))


def _tc_linear_gelu(emb, wt, bias):
    BM = 1024
    grid = (N_LOOKUPS // BM,)
    wt = wt.astype(jnp.bfloat16)
    return pl.pallas_call(
        _mm_body,
        grid=grid,
        in_specs=[
            pl.BlockSpec((BM, D), lambda i: (i, 0)),
            pl.BlockSpec((D, D), lambda i: (0, 0)),
            pl.BlockSpec((1, D), lambda i: (0, 0)),
        ],
        out_specs=pl.BlockSpec((BM, D), lambda i: (i, 0)),
        out_shape=jax.ShapeDtypeStruct((N_LOOKUPS, D), jnp.float32),
    )(emb, wt, bias)


def kernel(x, table, W, b):
    x_t = x.astype(jnp.int32).T.reshape(-1)
    emb = _sc_gather(x_t, table)
    out = _tc_linear_gelu(emb, W.T, b.reshape(1, D))
    return out.reshape(SEQ, B_ROWS, D).transpose(1, 0, 2)


# speedup vs baseline: 6.6886x; 1.0245x over previous
import functools

import jax
import jax.numpy as jnp
from jax import lax
from jax.experimental import pallas as pl
from jax.experimental.pallas import tpu as pltpu
from jax.experimental.pallas import tpu_sc as plsc

B_ROWS = 4096
SEQ = 20
D = 768
N_LOOKUPS = B_ROWS * SEQ

NC = 2
NS = 16
NW = NC * NS
CHUNK = 128

N_CH = 4
CH = N_LOOKUPS // N_CH
ROWS_PER_W = CH // NW
NCHUNK = ROWS_PER_W // CHUNK

BM = 1024
BLOCKS_PER_CH = CH // BM


def _gather_body(x_hbm, table_hbm, out_hbm, idx_v, rows_v, sem):
    wid = lax.axis_index("s") * NC + lax.axis_index("c")
    base = wid * ROWS_PER_W

    def chunk(k, carry):
        off = base + k * CHUNK
        pltpu.sync_copy(x_hbm.at[pl.ds(off, CHUNK)], idx_v)
        pltpu.async_copy(table_hbm.at[idx_v], rows_v, sem).wait()
        pltpu.sync_copy(rows_v, out_hbm.at[pl.ds(off, CHUNK)])
        return carry

    lax.fori_loop(0, NCHUNK, chunk, 0)


def _sc_gather(x_chunk, table):
    mesh = plsc.VectorSubcoreMesh(core_axis_name="c", subcore_axis_name="s")
    kern = functools.partial(
        pl.kernel,
        mesh=mesh,
        out_type=jax.ShapeDtypeStruct((CH, D), jnp.float32),
        scratch_types=[
            pltpu.VMEM((CHUNK,), jnp.int32),
            pltpu.VMEM((CHUNK, D), jnp.float32),
            pltpu.SemaphoreType.DMA,
        ],
    )(_gather_body)
    return kern(x_chunk, table)


_SQRT_HALF = 0.7071067811865476


def _mm_first_body(emb_ref, wt_ref, b_ref, out_ref):
    h = jnp.dot(emb_ref[...], wt_ref[...], preferred_element_type=jnp.float32)
    h = h + b_ref[...]
    out_ref[...] = 0.5 * h * (1.0 + lax.erf(h * _SQRT_HALF))


def _mm_alias_body(buf_ref, emb_ref, wt_ref, b_ref, out_ref):
    del buf_ref
    h = jnp.dot(emb_ref[...], wt_ref[...], preferred_element_type=jnp.float32)
    h = h + b_ref[...]
    out_ref[...] = 0.5 * h * (1.0 + lax.erf(h * _SQRT_HALF))


def _tc_chunk(buf, emb_c, wt, bias, c):
    c_off = c * BLOCKS_PER_CH
    out_spec = pl.BlockSpec((BM, D), lambda i, _c=c_off: (i + _c, 0))
    emb_spec = pl.BlockSpec((BM, D), lambda i: (i, 0))
    wt_spec = pl.BlockSpec((D, D), lambda i: (0, 0))
    b_spec = pl.BlockSpec((1, D), lambda i: (0, 0))
    out_shape = jax.ShapeDtypeStruct((N_LOOKUPS, D), jnp.float32)
    if buf is None:
        return pl.pallas_call(
            _mm_first_body,
            grid=(BLOCKS_PER_CH,),
            in_specs=[emb_spec, wt_spec, b_spec],
            out_specs=out_spec,
            out_shape=out_shape,
        )(emb_c, wt, bias)
    return pl.pallas_call(
        _mm_alias_body,
        grid=(BLOCKS_PER_CH,),
        in_specs=[pl.BlockSpec(memory_space=pl.ANY),
                  emb_spec, wt_spec, b_spec],
        out_specs=out_spec,
        out_shape=out_shape,
        input_output_aliases={0: 0},
    )(buf, emb_c, wt, bias)


def kernel(x, table, W, b):
    x_t = x.astype(jnp.int32).T.reshape(-1)
    wt = W.T
    bias = b.reshape(1, D)
    embs = [_sc_gather(lax.slice(x_t, (c * CH,), ((c + 1) * CH,)), table)
            for c in range(N_CH)]
    buf = None
    for c in range(N_CH):
        buf = _tc_chunk(buf, embs[c], wt, bias, c)
    return buf.reshape(SEQ, B_ROWS, D).transpose(1, 0, 2)


# speedup vs baseline: 7.0397x; 1.0525x over previous
import functools

import jax
import jax.numpy as jnp
from jax import lax
from jax.experimental import pallas as pl
from jax.experimental.pallas import tpu as pltpu
from jax.experimental.pallas import tpu_sc as plsc

B_ROWS = 4096
SEQ = 20
D = 768
N_LOOKUPS = B_ROWS * SEQ

NC = 2
NS = 16
NW = NC * NS
CHUNK = 64

N_CH = 4
CH = N_LOOKUPS // N_CH
ROWS_PER_W = CH // NW
NCHUNK = ROWS_PER_W // CHUNK


BM = 1024
BLOCKS_PER_CH = CH // BM


def _gather_body(x_hbm, table_hbm, out_hbm, idx_v, rows0, rows1, sem0, sem1):
    wid = lax.axis_index("s") * NC + lax.axis_index("c")
    base = wid * ROWS_PER_W
    bufs = (rows0, rows1)
    sems = (sem0, sem1)

    def gather_dma(k, p):
        return pltpu.make_async_copy(
            table_hbm.at[idx_v.at[pl.ds(k * CHUNK, CHUNK)]], bufs[p], sems[p])

    pltpu.sync_copy(x_hbm.at[pl.ds(base, ROWS_PER_W)], idx_v)
    gather_dma(0, 0).start()

    def pair(i, carry):
        for p in range(2):
            k = i * 2 + p

            @pl.when(k + 1 < NCHUNK)
            def _():
                gather_dma(k + 1, (p + 1) % 2).start()

            gather_dma(k, p).wait()
            pltpu.sync_copy(bufs[p], out_hbm.at[pl.ds(base + k * CHUNK, CHUNK)])
        return carry

    lax.fori_loop(0, NCHUNK // 2, pair, 0)


def _sc_gather(x_chunk, table):
    mesh = plsc.VectorSubcoreMesh(core_axis_name="c", subcore_axis_name="s")
    kern = functools.partial(
        pl.kernel,
        mesh=mesh,
        out_type=jax.ShapeDtypeStruct((CH, D), jnp.float32),
        scratch_types=[
            pltpu.VMEM((ROWS_PER_W,), jnp.int32),
            pltpu.VMEM((CHUNK, D), jnp.float32),
            pltpu.VMEM((CHUNK, D), jnp.float32),
            pltpu.SemaphoreType.DMA,
            pltpu.SemaphoreType.DMA,
        ],
    )(_gather_body)
    return kern(x_chunk, table)


_SQRT_HALF = 0.7071067811865476


def _mm_first_body(emb_ref, wt_ref, b_ref, out_ref):
    h = jnp.dot(emb_ref[...], wt_ref[...], preferred_element_type=jnp.float32)
    h = h + b_ref[...]
    out_ref[...] = 0.5 * h * (1.0 + lax.erf(h * _SQRT_HALF))


def _mm_alias_body(buf_ref, emb_ref, wt_ref, b_ref, out_ref):
    del buf_ref
    h = jnp.dot(emb_ref[...], wt_ref[...], preferred_element_type=jnp.float32)
    h = h + b_ref[...]
    out_ref[...] = 0.5 * h * (1.0 + lax.erf(h * _SQRT_HALF))


def _tc_chunk(buf, emb_c, wt, bias, c):
    c_off = c * BLOCKS_PER_CH
    out_spec = pl.BlockSpec((BM, D), lambda i, _c=c_off: (i + _c, 0))
    emb_spec = pl.BlockSpec((BM, D), lambda i: (i, 0))
    wt_spec = pl.BlockSpec((D, D), lambda i: (0, 0))
    b_spec = pl.BlockSpec((1, D), lambda i: (0, 0))
    out_shape = jax.ShapeDtypeStruct((N_LOOKUPS, D), jnp.float32)
    if buf is None:
        return pl.pallas_call(
            _mm_first_body,
            grid=(BLOCKS_PER_CH,),
            in_specs=[emb_spec, wt_spec, b_spec],
            out_specs=out_spec,
            out_shape=out_shape,
        )(emb_c, wt, bias)
    return pl.pallas_call(
        _mm_alias_body,
        grid=(BLOCKS_PER_CH,),
        in_specs=[pl.BlockSpec(memory_space=pl.ANY),
                  emb_spec, wt_spec, b_spec],
        out_specs=out_spec,
        out_shape=out_shape,
        input_output_aliases={0: 0},
    )(buf, emb_c, wt, bias)


def kernel(x, table, W, b):
    x_t = x.astype(jnp.int32).T.reshape(-1)
    wt = W.T
    bias = b.reshape(1, D)
    embs = [_sc_gather(lax.slice(x_t, (c * CH,), ((c + 1) * CH,)), table)
            for c in range(N_CH)]
    buf = None
    for c in range(N_CH):
        buf = _tc_chunk(buf, embs[c], wt, bias, c)
    return buf.reshape(SEQ, B_ROWS, D).transpose(1, 0, 2)
